# Initial kernel scaffold; baseline (speedup 1.0000x reference)
import jax
import jax.numpy as jnp
from jax.experimental import pallas as pl
from jax.experimental.pallas import tpu as pltpu

B, IMG, TXT, D = 8, 4096, 512, 2048
BLK = 512


def _fused_body(mask_ref, enc_ref, ts_ref, hs_ref, gamma_ref, tproj_ref,
                out_ref, mean_ref):
    j = pl.program_id(1)

    @pl.when(j == 0)
    def _compute_mean():
        m = mask_ref[0, 0, :] > -9000.0
        cnt = jnp.sum(m.astype(jnp.float32))
        keepf = m.astype(jnp.float32)[:, None]
        s = jnp.sum(enc_ref[0] * keepf, axis=0)
        t_emb = ts_ref[0, 0] * tproj_ref[0, :] * 0.001
        mean_ref[0, :] = s / jnp.maximum(cnt, 1.0) + t_emb

    add = mean_ref[0, :][None, :]
    out_ref[0] = hs_ref[0] * gamma_ref[0, :][None, :] + add


@jax.jit
def _run(hidden_states, encoder_hidden_states, encoder_attention_mask,
         gamma, t_proj, ts_f):
    grid = (B, IMG // BLK)
    return pl.pallas_call(
        _fused_body,
        grid=grid,
        in_specs=[
            pl.BlockSpec((1, 1, TXT), lambda b, j: (b, 0, 0)),
            pl.BlockSpec((1, TXT, D), lambda b, j: (b, 0, 0)),
            pl.BlockSpec((1, 1), lambda b, j: (b, 0)),
            pl.BlockSpec((1, BLK, D), lambda b, j: (b, j, 0)),
            pl.BlockSpec((1, D), lambda b, j: (0, 0)),
            pl.BlockSpec((1, D), lambda b, j: (0, 0)),
        ],
        out_specs=pl.BlockSpec((1, BLK, D), lambda b, j: (b, j, 0)),
        out_shape=jax.ShapeDtypeStruct((B, IMG, D), jnp.float32),
        scratch_shapes=[pltpu.VMEM((1, D), jnp.float32)],
        compiler_params=pltpu.CompilerParams(
            dimension_semantics=("arbitrary", "arbitrary"),
        ),
    )(encoder_attention_mask, encoder_hidden_states, ts_f,
      hidden_states, gamma.reshape(1, D), t_proj.reshape(1, D))


def kernel(hidden_states, encoder_hidden_states, encoder_attention_mask,
           gamma, t_proj, timestep):
    ts_f = timestep.astype(jnp.float32).reshape(B, 1)
    return _run(hidden_states, encoder_hidden_states, encoder_attention_mask,
                gamma, t_proj, ts_f)


# speedup vs baseline: 1.2084x; 1.2084x over previous
import jax
import jax.numpy as jnp
from jax.experimental import pallas as pl
from jax.experimental.pallas import tpu as pltpu

B, IMG, TXT, D = 8, 4096, 512, 2048
BLK = 512


def _fused_body(mask_ref, enc_ref, ts_ref, hs_ref, gamma_ref, tproj_ref,
                out_ref, mean_ref):
    j = pl.program_id(1)

    @pl.when(j == 0)
    def _compute_mean():
        m = mask_ref[0, 0, :] > -9000.0
        cnt = jnp.sum(m.astype(jnp.float32))
        keepf = m.astype(jnp.float32)[:, None]
        s = jnp.sum(enc_ref[0] * keepf, axis=0)
        t_emb = ts_ref[0, 0, 0] * tproj_ref[0, :] * 0.001
        mean_ref[0, :] = s / jnp.maximum(cnt, 1.0) + t_emb

    add = mean_ref[0, :][None, :]
    out_ref[0] = hs_ref[0] * gamma_ref[0, :][None, :] + add


@jax.jit
def _run(hidden_states, encoder_hidden_states, encoder_attention_mask,
         gamma, t_proj, ts_f):
    grid = (B, IMG // BLK)
    return pl.pallas_call(
        _fused_body,
        grid=grid,
        in_specs=[
            pl.BlockSpec((1, 1, TXT), lambda b, j: (b, 0, 0)),
            pl.BlockSpec((1, TXT, D), lambda b, j: (b, 0, 0)),
            pl.BlockSpec((1, 1, 1), lambda b, j: (b, 0, 0)),
            pl.BlockSpec((1, BLK, D), lambda b, j: (b, j, 0)),
            pl.BlockSpec((1, D), lambda b, j: (0, 0)),
            pl.BlockSpec((1, D), lambda b, j: (0, 0)),
        ],
        out_specs=pl.BlockSpec((1, BLK, D), lambda b, j: (b, j, 0)),
        out_shape=jax.ShapeDtypeStruct((B, IMG, D), jnp.float32),
        scratch_shapes=[pltpu.VMEM((1, D), jnp.float32)],
        compiler_params=pltpu.CompilerParams(
            dimension_semantics=("arbitrary", "arbitrary"),
        ),
    )(encoder_attention_mask, encoder_hidden_states, ts_f,
      hidden_states, gamma.reshape(1, D), t_proj.reshape(1, D))


def kernel(hidden_states, encoder_hidden_states, encoder_attention_mask,
           gamma, t_proj, timestep):
    ts_f = timestep.astype(jnp.float32).reshape(B, 1, 1)
    return _run(hidden_states, encoder_hidden_states, encoder_attention_mask,
                gamma, t_proj, ts_f)


# speedup vs baseline: 1.2470x; 1.0320x over previous
import jax
import jax.numpy as jnp
from jax.experimental import pallas as pl
from jax.experimental.pallas import tpu as pltpu

B, IMG, TXT, D = 8, 4096, 512, 2048
BLK = 1024


def _fused_body(mask_ref, enc_ref, ts_ref, hs_ref, gamma_ref, tproj_ref,
                out_ref, mean_ref):
    j = pl.program_id(1)

    @pl.when(j == 0)
    def _compute_mean():
        m = mask_ref[0, 0, :] > -9000.0
        cnt = jnp.sum(m.astype(jnp.float32))
        keepf = m.astype(jnp.float32)[:, None]
        s = jnp.sum(enc_ref[0] * keepf, axis=0)
        t_emb = ts_ref[0, 0, 0] * tproj_ref[0, :] * 0.001
        mean_ref[0, :] = s / jnp.maximum(cnt, 1.0) + t_emb

    add = mean_ref[0, :][None, :]
    out_ref[0] = hs_ref[0] * gamma_ref[0, :][None, :] + add


@jax.jit
def _run(hidden_states, encoder_hidden_states, encoder_attention_mask,
         gamma, t_proj, ts_f):
    grid = (B, IMG // BLK)
    return pl.pallas_call(
        _fused_body,
        grid=grid,
        in_specs=[
            pl.BlockSpec((1, 1, TXT), lambda b, j: (b, 0, 0)),
            pl.BlockSpec((1, TXT, D), lambda b, j: (b, 0, 0)),
            pl.BlockSpec((1, 1, 1), lambda b, j: (b, 0, 0)),
            pl.BlockSpec((1, BLK, D), lambda b, j: (b, j, 0)),
            pl.BlockSpec((1, D), lambda b, j: (0, 0)),
            pl.BlockSpec((1, D), lambda b, j: (0, 0)),
        ],
        out_specs=pl.BlockSpec((1, BLK, D), lambda b, j: (b, j, 0)),
        out_shape=jax.ShapeDtypeStruct((B, IMG, D), jnp.float32),
        scratch_shapes=[pltpu.VMEM((1, D), jnp.float32)],
        compiler_params=pltpu.CompilerParams(
            dimension_semantics=("arbitrary", "arbitrary"),
        ),
    )(encoder_attention_mask, encoder_hidden_states, ts_f,
      hidden_states, gamma.reshape(1, D), t_proj.reshape(1, D))


def kernel(hidden_states, encoder_hidden_states, encoder_attention_mask,
           gamma, t_proj, timestep):
    ts_f = timestep.astype(jnp.float32).reshape(B, 1, 1)
    return _run(hidden_states, encoder_hidden_states, encoder_attention_mask,
                gamma, t_proj, ts_f)


# speedup vs baseline: 1.2490x; 1.0016x over previous
import jax
import jax.numpy as jnp
from jax.experimental import pallas as pl
from jax.experimental.pallas import tpu as pltpu

B, IMG, TXT, D = 8, 4096, 512, 2048
BLK = 1024


def _fused_body(mask_ref, enc_ref, ts_ref, hs_ref, gamma_ref, tproj_ref,
                out_ref, mean_ref):
    j = pl.program_id(1)

    @pl.when(j == 0)
    def _compute_mean():
        m = mask_ref[0, 0, :] > -9000.0
        cnt = jnp.sum(m.astype(jnp.float32))
        keepf = m.astype(jnp.float32)[:, None]
        s = jnp.sum(enc_ref[0] * keepf, axis=0)
        t_emb = ts_ref[0, 0, 0] * tproj_ref[0, :] * 0.001
        mean_ref[0, :] = s / jnp.maximum(cnt, 1.0) + t_emb

    add = mean_ref[0, :][None, :]
    out_ref[0] = hs_ref[0] * gamma_ref[0, :][None, :] + add


@jax.jit
def _run(hidden_states, encoder_hidden_states, encoder_attention_mask,
         gamma, t_proj, ts_f):
    grid = (B, IMG // BLK)
    return pl.pallas_call(
        _fused_body,
        grid=grid,
        in_specs=[
            pl.BlockSpec((1, 1, TXT), lambda b, j: (b, 0, 0)),
            pl.BlockSpec((1, TXT, D), lambda b, j: (b, 0, 0)),
            pl.BlockSpec((1, 1, 1), lambda b, j: (b, 0, 0)),
            pl.BlockSpec((1, BLK, D), lambda b, j: (b, j, 0)),
            pl.BlockSpec((1, D), lambda b, j: (0, 0)),
            pl.BlockSpec((1, D), lambda b, j: (0, 0)),
        ],
        out_specs=pl.BlockSpec((1, BLK, D), lambda b, j: (b, j, 0)),
        out_shape=jax.ShapeDtypeStruct((B, IMG, D), jnp.float32),
        scratch_shapes=[pltpu.VMEM((1, D), jnp.float32)],
        compiler_params=pltpu.CompilerParams(
            dimension_semantics=("parallel", "arbitrary"),
        ),
    )(encoder_attention_mask, encoder_hidden_states, ts_f,
      hidden_states, gamma.reshape(1, D), t_proj.reshape(1, D))


def kernel(hidden_states, encoder_hidden_states, encoder_attention_mask,
           gamma, t_proj, timestep):
    ts_f = timestep.astype(jnp.float32).reshape(B, 1, 1)
    return _run(hidden_states, encoder_hidden_states, encoder_attention_mask,
                gamma, t_proj, ts_f)
